# baseline (speedup 1.0000x reference)
import functools

import jax
import jax.numpy as jnp
from jax import lax
from jax.experimental import pallas as pl
from jax.experimental.pallas import tpu as pltpu
from jax.experimental.pallas import tpu_sc as plsc

N = 10000
E = 160000
D = 256
H = 16
C = 64

BLK = 1000
NACC = 10240
KB = 128
NT = 32
EPT = 5120
NCHUNK = EPT // KB
RPT = NACC // 16
EPAD = NT * EPT
NBUF = 4


def _sc_mesh():
    return plsc.VectorSubcoreMesh(core_axis_name="c", subcore_axis_name="s")


def _deg_body(ewb, dstb, zeros_hbm, out_hbm, dst_v, ew_v, acc, sem):
    c = lax.axis_index("c")
    s = lax.axis_index("s")
    wid = s * 2 + c
    pltpu.sync_copy(dstb.at[wid], dst_v)
    pltpu.sync_copy(ewb.at[wid], ew_v)
    base = s * RPT
    pltpu.sync_copy(zeros_hbm.at[pl.ds(base, RPT)], acc.at[pl.ds(base, RPT)])
    plsc.subcore_barrier()

    def fire(j, carry):
        pltpu.async_copy(ew_v.at[j], acc.at[dst_v.at[j]], sem, add=True)
        return carry

    lax.fori_loop(0, NCHUNK, fire, 0)

    def drain(j, carry):
        pltpu.make_async_copy(ew_v.at[j], acc.at[dst_v.at[j]], sem).wait()
        return carry

    lax.fori_loop(0, NCHUNK, drain, 0)
    plsc.subcore_barrier()
    pltpu.sync_copy(acc.at[pl.ds(base, RPT)], out_hbm.at[c, pl.ds(base, RPT)])


_deg_kernel = functools.partial(
    pl.kernel,
    out_type=jax.ShapeDtypeStruct((2, NACC), jnp.float32),
    mesh=_sc_mesh(),
    scratch_types=[
        pltpu.VMEM((NCHUNK, KB), jnp.int32),
        pltpu.VMEM((NCHUNK, KB), jnp.float32),
        pltpu.VMEM_SHARED((NACC,), jnp.float32),
        pltpu.SemaphoreType.DMA,
    ],
    compiler_params=pltpu.CompilerParams(use_tc_tiling_on_sc=False),
)(_deg_body)


def _make_agg(F):
    def body(table_hbm, srcb, dstb, ewb, out_hbm, src_v, dst_v, ew_v, rows_v,
             acc, *sems):
        gsems = sems[:NBUF]
        ssems = sems[NBUF:]
        c = lax.axis_index("c")
        s = lax.axis_index("s")
        wid = s * 2 + c
        pltpu.sync_copy(srcb.at[wid], src_v)
        pltpu.sync_copy(dstb.at[wid], dst_v)
        pltpu.sync_copy(ewb.at[wid], ew_v)
        base = s * RPT
        pltpu.sync_copy(table_hbm.at[pl.ds(base, RPT)], acc.at[pl.ds(base, RPT)])
        plsc.subcore_barrier()

        def issue_gather(j, b):
            pltpu.async_copy(table_hbm.at[src_v.at[j]], rows_v.at[b], gsems[b])

        def wait_gather(j, b):
            pltpu.make_async_copy(table_hbm.at[src_v.at[j]], rows_v.at[b],
                                  gsems[b]).wait()

        def issue_scatter(j, b):
            pltpu.async_copy(rows_v.at[b], acc.at[dst_v.at[j]], ssems[b],
                             add=True)

        def wait_scatter(j, b):
            pltpu.make_async_copy(rows_v.at[b], acc.at[dst_v.at[j]],
                                  ssems[b]).wait()

        def scale(j, b):
            def sbody(g, carry):
                wv = ew_v[j, pl.ds(g * 16, 16)]
                for k in range(16):
                    w = wv[k]
                    i = g * 16 + k
                    for f in range(F // 16):
                        sl = pl.ds(f * 16, 16)
                        rows_v[b, i, sl] = rows_v[b, i, sl] * w
                return carry

            lax.fori_loop(0, KB // 16, sbody, 0)

        for b in range(NBUF - 1):
            issue_gather(b, b)

        def step(j, b, bnext):
            wait_gather(j, b)
            scale(j, b)

            @pl.when(j > 0)
            def _():
                wait_scatter(j - 1, bnext)

            issue_scatter(j, b)

            @pl.when(j + NBUF - 1 < NCHUNK)
            def _():
                issue_gather(j + NBUF - 1, bnext)

        def body2(t, carry):
            for u in range(NBUF):
                step(NBUF * t + u, u, (u + NBUF - 1) % NBUF)
            return carry

        lax.fori_loop(0, NCHUNK // NBUF, body2, 0)
        wait_scatter(NCHUNK - 1, (NCHUNK - 1) % NBUF)
        plsc.subcore_barrier()
        pltpu.sync_copy(acc.at[pl.ds(base, RPT)],
                        out_hbm.at[c, pl.ds(base, RPT)])

    return functools.partial(
        pl.kernel,
        out_type=jax.ShapeDtypeStruct((2, NACC, F), jnp.float32),
        mesh=_sc_mesh(),
        scratch_types=[
            pltpu.VMEM((NCHUNK, KB), jnp.int32),
            pltpu.VMEM((NCHUNK, KB), jnp.int32),
            pltpu.VMEM((NCHUNK, KB), jnp.float32),
            pltpu.VMEM((NBUF, KB, F), jnp.float32),
            pltpu.VMEM_SHARED((NACC, F), jnp.float32),
        ] + [pltpu.SemaphoreType.DMA] * (2 * NBUF),
        compiler_params=pltpu.CompilerParams(use_tc_tiling_on_sc=False),
    )(body)


_agg16 = _make_agg(H)
_agg64 = _make_agg(C)


def _mm1_body(x_ref, w_ref, xw_ref):
    xw_ref[...] = jnp.dot(x_ref[...], w_ref[...],
                          preferred_element_type=jnp.float32)


def _tc_mm1(x, W1):
    return pl.pallas_call(
        _mm1_body,
        grid=(N // BLK,),
        in_specs=[
            pl.BlockSpec((BLK, D), lambda i: (i, 0)),
            pl.BlockSpec((D, H), lambda i: (0, 0)),
        ],
        out_specs=pl.BlockSpec((BLK, H), lambda i: (i, 0)),
        out_shape=jax.ShapeDtypeStruct((N, H), jnp.float32),
    )(x, W1)


def _scale1_body(xw_ref, d0_ref, d1_ref, xs_ref, dinv_ref):
    dinv = lax.rsqrt(1.0 + d0_ref[0] + d1_ref[0])
    xs_ref[...] = xw_ref[...] * dinv
    dinv_ref[...] = dinv


def _tc_scale1(xw1, degp):
    return pl.pallas_call(
        _scale1_body,
        grid=(N // BLK,),
        in_specs=[
            pl.BlockSpec((BLK, H), lambda i: (i, 0)),
            pl.BlockSpec((1, BLK, 1), lambda i: (0, i, 0)),
            pl.BlockSpec((1, BLK, 1), lambda i: (1, i, 0)),
        ],
        out_specs=[
            pl.BlockSpec((BLK, H), lambda i: (i, 0)),
            pl.BlockSpec((BLK, 1), lambda i: (i, 0)),
        ],
        out_shape=[
            jax.ShapeDtypeStruct((N, H), jnp.float32),
            jax.ShapeDtypeStruct((N, 1), jnp.float32),
        ],
    )(xw1, degp, degp)


def _mm2_body(p0_ref, p1_ref, xs1_ref, dinv_ref, b1_ref, w2_ref, xs2_ref):
    dinv = dinv_ref[...]
    h = (p0_ref[0] + p1_ref[0] - xs1_ref[...]) * dinv + b1_ref[...]
    h = jnp.maximum(h, 0.0)
    xw2 = jnp.dot(h, w2_ref[...], preferred_element_type=jnp.float32)
    xs2_ref[...] = xw2 * dinv


def _tc_stage2(p, xs1, dinv, b1, W2):
    return pl.pallas_call(
        _mm2_body,
        grid=(N // BLK,),
        in_specs=[
            pl.BlockSpec((1, BLK, H), lambda i: (0, i, 0)),
            pl.BlockSpec((1, BLK, H), lambda i: (1, i, 0)),
            pl.BlockSpec((BLK, H), lambda i: (i, 0)),
            pl.BlockSpec((BLK, 1), lambda i: (i, 0)),
            pl.BlockSpec((1, H), lambda i: (0, 0)),
            pl.BlockSpec((H, C), lambda i: (0, 0)),
        ],
        out_specs=pl.BlockSpec((BLK, C), lambda i: (i, 0)),
        out_shape=jax.ShapeDtypeStruct((N, C), jnp.float32),
    )(p, p, xs1, dinv, b1, W2)


def _final_body(q0_ref, q1_ref, xs2_ref, dinv_ref, b2_ref, o_ref):
    o = (q0_ref[0] + q1_ref[0] - xs2_ref[...]) * dinv_ref[...] + b2_ref[...]
    m = jnp.max(o, axis=1, keepdims=True)
    ex = jnp.exp(o - m)
    sden = jnp.sum(ex, axis=1, keepdims=True)
    o_ref[...] = o - m - jnp.log(sden)


def _tc_final(q, xs2, dinv, b2):
    return pl.pallas_call(
        _final_body,
        grid=(N // BLK,),
        in_specs=[
            pl.BlockSpec((1, BLK, C), lambda i: (0, i, 0)),
            pl.BlockSpec((1, BLK, C), lambda i: (1, i, 0)),
            pl.BlockSpec((BLK, C), lambda i: (i, 0)),
            pl.BlockSpec((BLK, 1), lambda i: (i, 0)),
            pl.BlockSpec((1, C), lambda i: (0, 0)),
        ],
        out_specs=pl.BlockSpec((BLK, C), lambda i: (i, 0)),
        out_shape=jax.ShapeDtypeStruct((N, C), jnp.float32),
    )(q, q, xs2, dinv, b2)


def kernel(x, edge_index, edge_weight, W1, b1, W2, b2):
    src = edge_index[0]
    dst = edge_index[1]
    ew = edge_weight

    pad = EPAD - E
    srcb = jnp.concatenate([src, jnp.full((pad,), N, jnp.int32)]).reshape(NT, NCHUNK, KB)
    dstb = jnp.concatenate([dst, jnp.full((pad,), N, jnp.int32)]).reshape(NT, NCHUNK, KB)
    ewb = jnp.concatenate([ew, jnp.zeros((pad,), jnp.float32)]).reshape(NT, NCHUNK, KB)

    zeros_n = jnp.zeros((NACC,), jnp.float32)
    degp = _deg_kernel(ewb, dstb, zeros_n)

    xw1 = _tc_mm1(x, W1)
    xs1, dinv = _tc_scale1(xw1, degp.reshape(2, NACC, 1))

    xs1p = jnp.zeros((NACC, H), jnp.float32).at[:N].set(xs1)
    p = _agg16(xs1p, srcb, dstb, ewb)
    xs2 = _tc_stage2(p, xs1, dinv, b1.reshape(1, H), W2)

    xs2p = jnp.zeros((NACC, C), jnp.float32).at[:N].set(xs2)
    q = _agg64(xs2p, srcb, dstb, ewb)
    return _tc_final(q, xs2, dinv, b2.reshape(1, C))


# speedup vs baseline: 1.4752x; 1.4752x over previous
import functools

import jax
import jax.numpy as jnp
from jax import lax
from jax.experimental import pallas as pl
from jax.experimental.pallas import tpu as pltpu
from jax.experimental.pallas import tpu_sc as plsc

N = 10000
E = 160000
D = 256
H = 16
C = 64

BLK = 1000
NACC = 10240
KB = 128
NT = 32
EPT = 5120
NCHUNK = EPT // KB
RPT = NACC // 16
EPAD = NT * EPT
NBUF = 4


def _sc_mesh():
    return plsc.VectorSubcoreMesh(core_axis_name="c", subcore_axis_name="s")


def _deg_body(ewb, dstb, zeros_hbm, out_hbm, dst_v, ew_v, acc, sem):
    c = lax.axis_index("c")
    s = lax.axis_index("s")
    wid = s * 2 + c
    pltpu.sync_copy(dstb.at[wid], dst_v)
    pltpu.sync_copy(ewb.at[wid], ew_v)
    base = s * RPT
    pltpu.sync_copy(zeros_hbm.at[pl.ds(base, RPT)], acc.at[pl.ds(base, RPT)])
    plsc.subcore_barrier()

    def fire(j, carry):
        pltpu.async_copy(ew_v.at[j], acc.at[dst_v.at[j]], sem, add=True)
        return carry

    lax.fori_loop(0, NCHUNK, fire, 0)

    def drain(j, carry):
        pltpu.make_async_copy(ew_v.at[j], acc.at[dst_v.at[j]], sem).wait()
        return carry

    lax.fori_loop(0, NCHUNK, drain, 0)
    plsc.subcore_barrier()
    pltpu.sync_copy(acc.at[pl.ds(base, RPT)], out_hbm.at[c, pl.ds(base, RPT)])


_deg_kernel = functools.partial(
    pl.kernel,
    out_type=jax.ShapeDtypeStruct((2, NACC), jnp.float32),
    mesh=_sc_mesh(),
    scratch_types=[
        pltpu.VMEM((NCHUNK, KB), jnp.int32),
        pltpu.VMEM((NCHUNK, KB), jnp.float32),
        pltpu.VMEM_SHARED((NACC,), jnp.float32),
        pltpu.SemaphoreType.DMA,
    ],
    compiler_params=pltpu.CompilerParams(use_tc_tiling_on_sc=False),
)(_deg_body)


def _make_agg(F):
    def body(table_hbm, srcb, dstb, ewb, out_hbm, src_v, dst_v, ew_v, rows_v,
             acc, table_s, *sems):
        gsems = sems[:NBUF]
        ssems = sems[NBUF:]
        c = lax.axis_index("c")
        s = lax.axis_index("s")
        wid = s * 2 + c
        pltpu.sync_copy(srcb.at[wid], src_v)
        pltpu.sync_copy(dstb.at[wid], dst_v)
        pltpu.sync_copy(ewb.at[wid], ew_v)
        base = s * RPT
        pltpu.sync_copy(table_hbm.at[pl.ds(base, RPT)], acc.at[pl.ds(base, RPT)])
        pltpu.sync_copy(table_hbm.at[pl.ds(base, RPT)],
                        table_s.at[pl.ds(base, RPT)])
        plsc.subcore_barrier()

        def issue_gather(j, b):
            pltpu.async_copy(table_s.at[src_v.at[j]], rows_v.at[b], gsems[b])

        def wait_gather(j, b):
            pltpu.make_async_copy(table_s.at[src_v.at[j]], rows_v.at[b],
                                  gsems[b]).wait()

        def issue_scatter(j, b):
            pltpu.async_copy(rows_v.at[b], acc.at[dst_v.at[j]], ssems[b],
                             add=True)

        def wait_scatter(j, b):
            pltpu.make_async_copy(rows_v.at[b], acc.at[dst_v.at[j]],
                                  ssems[b]).wait()

        def scale(j, b):
            def sbody(g, carry):
                wv = ew_v[j, pl.ds(g * 16, 16)]
                for k in range(16):
                    w = wv[k]
                    i = g * 16 + k
                    for f in range(F // 16):
                        sl = pl.ds(f * 16, 16)
                        rows_v[b, i, sl] = rows_v[b, i, sl] * w
                return carry

            lax.fori_loop(0, KB // 16, sbody, 0)

        for b in range(NBUF - 1):
            issue_gather(b, b)

        def step(j, b, bnext):
            wait_gather(j, b)
            scale(j, b)

            @pl.when(j > 0)
            def _():
                wait_scatter(j - 1, bnext)

            issue_scatter(j, b)

            @pl.when(j + NBUF - 1 < NCHUNK)
            def _():
                issue_gather(j + NBUF - 1, bnext)

        def body2(t, carry):
            for u in range(NBUF):
                step(NBUF * t + u, u, (u + NBUF - 1) % NBUF)
            return carry

        lax.fori_loop(0, NCHUNK // NBUF, body2, 0)
        wait_scatter(NCHUNK - 1, (NCHUNK - 1) % NBUF)
        plsc.subcore_barrier()
        pltpu.sync_copy(acc.at[pl.ds(base, RPT)],
                        out_hbm.at[c, pl.ds(base, RPT)])

    return functools.partial(
        pl.kernel,
        out_type=jax.ShapeDtypeStruct((2, NACC, F), jnp.float32),
        mesh=_sc_mesh(),
        scratch_types=[
            pltpu.VMEM((NCHUNK, KB), jnp.int32),
            pltpu.VMEM((NCHUNK, KB), jnp.int32),
            pltpu.VMEM((NCHUNK, KB), jnp.float32),
            pltpu.VMEM((NBUF, KB, F), jnp.float32),
            pltpu.VMEM_SHARED((NACC, F), jnp.float32),
            pltpu.VMEM_SHARED((NACC, F), jnp.float32),
        ] + [pltpu.SemaphoreType.DMA] * (2 * NBUF),
        compiler_params=pltpu.CompilerParams(use_tc_tiling_on_sc=False),
    )(body)


_agg16 = _make_agg(H)
_agg64 = _make_agg(C)


def _mm1_body(x_ref, w_ref, xw_ref):
    xw_ref[...] = jnp.dot(x_ref[...], w_ref[...],
                          preferred_element_type=jnp.float32)


def _tc_mm1(x, W1):
    return pl.pallas_call(
        _mm1_body,
        grid=(N // BLK,),
        in_specs=[
            pl.BlockSpec((BLK, D), lambda i: (i, 0)),
            pl.BlockSpec((D, H), lambda i: (0, 0)),
        ],
        out_specs=pl.BlockSpec((BLK, H), lambda i: (i, 0)),
        out_shape=jax.ShapeDtypeStruct((N, H), jnp.float32),
    )(x, W1)


def _scale1_body(xw_ref, d0_ref, d1_ref, xs_ref, dinv_ref):
    dinv = lax.rsqrt(1.0 + d0_ref[0] + d1_ref[0])
    xs_ref[...] = xw_ref[...] * dinv
    dinv_ref[...] = dinv


def _tc_scale1(xw1, degp):
    return pl.pallas_call(
        _scale1_body,
        grid=(N // BLK,),
        in_specs=[
            pl.BlockSpec((BLK, H), lambda i: (i, 0)),
            pl.BlockSpec((1, BLK, 1), lambda i: (0, i, 0)),
            pl.BlockSpec((1, BLK, 1), lambda i: (1, i, 0)),
        ],
        out_specs=[
            pl.BlockSpec((BLK, H), lambda i: (i, 0)),
            pl.BlockSpec((BLK, 1), lambda i: (i, 0)),
        ],
        out_shape=[
            jax.ShapeDtypeStruct((N, H), jnp.float32),
            jax.ShapeDtypeStruct((N, 1), jnp.float32),
        ],
    )(xw1, degp, degp)


def _mm2_body(p0_ref, p1_ref, xs1_ref, dinv_ref, b1_ref, w2_ref, xs2_ref):
    dinv = dinv_ref[...]
    h = (p0_ref[0] + p1_ref[0] - xs1_ref[...]) * dinv + b1_ref[...]
    h = jnp.maximum(h, 0.0)
    xw2 = jnp.dot(h, w2_ref[...], preferred_element_type=jnp.float32)
    xs2_ref[...] = xw2 * dinv


def _tc_stage2(p, xs1, dinv, b1, W2):
    return pl.pallas_call(
        _mm2_body,
        grid=(N // BLK,),
        in_specs=[
            pl.BlockSpec((1, BLK, H), lambda i: (0, i, 0)),
            pl.BlockSpec((1, BLK, H), lambda i: (1, i, 0)),
            pl.BlockSpec((BLK, H), lambda i: (i, 0)),
            pl.BlockSpec((BLK, 1), lambda i: (i, 0)),
            pl.BlockSpec((1, H), lambda i: (0, 0)),
            pl.BlockSpec((H, C), lambda i: (0, 0)),
        ],
        out_specs=pl.BlockSpec((BLK, C), lambda i: (i, 0)),
        out_shape=jax.ShapeDtypeStruct((N, C), jnp.float32),
    )(p, p, xs1, dinv, b1, W2)


def _final_body(q0_ref, q1_ref, xs2_ref, dinv_ref, b2_ref, o_ref):
    o = (q0_ref[0] + q1_ref[0] - xs2_ref[...]) * dinv_ref[...] + b2_ref[...]
    m = jnp.max(o, axis=1, keepdims=True)
    ex = jnp.exp(o - m)
    sden = jnp.sum(ex, axis=1, keepdims=True)
    o_ref[...] = o - m - jnp.log(sden)


def _tc_final(q, xs2, dinv, b2):
    return pl.pallas_call(
        _final_body,
        grid=(N // BLK,),
        in_specs=[
            pl.BlockSpec((1, BLK, C), lambda i: (0, i, 0)),
            pl.BlockSpec((1, BLK, C), lambda i: (1, i, 0)),
            pl.BlockSpec((BLK, C), lambda i: (i, 0)),
            pl.BlockSpec((BLK, 1), lambda i: (i, 0)),
            pl.BlockSpec((1, C), lambda i: (0, 0)),
        ],
        out_specs=pl.BlockSpec((BLK, C), lambda i: (i, 0)),
        out_shape=jax.ShapeDtypeStruct((N, C), jnp.float32),
    )(q, q, xs2, dinv, b2)


def kernel(x, edge_index, edge_weight, W1, b1, W2, b2):
    src = edge_index[0]
    dst = edge_index[1]
    ew = edge_weight

    pad = EPAD - E
    srcb = jnp.concatenate([src, jnp.full((pad,), N, jnp.int32)]).reshape(NT, NCHUNK, KB)
    dstb = jnp.concatenate([dst, jnp.full((pad,), N, jnp.int32)]).reshape(NT, NCHUNK, KB)
    ewb = jnp.concatenate([ew, jnp.zeros((pad,), jnp.float32)]).reshape(NT, NCHUNK, KB)

    zeros_n = jnp.zeros((NACC,), jnp.float32)
    degp = _deg_kernel(ewb, dstb, zeros_n)

    xw1 = _tc_mm1(x, W1)
    xs1, dinv = _tc_scale1(xw1, degp.reshape(2, NACC, 1))

    xs1p = jnp.zeros((NACC, H), jnp.float32).at[:N].set(xs1)
    p = _agg16(xs1p, srcb, dstb, ewb)
    xs2 = _tc_stage2(p, xs1, dinv, b1.reshape(1, H), W2)

    xs2p = jnp.zeros((NACC, C), jnp.float32).at[:N].set(xs2)
    q = _agg64(xs2p, srcb, dstb, ewb)
    return _tc_final(q, xs2, dinv, b2.reshape(1, C))


# speedup vs baseline: 1.5672x; 1.0624x over previous
import functools

import jax
import jax.numpy as jnp
from jax import lax
from jax.experimental import pallas as pl
from jax.experimental.pallas import tpu as pltpu
from jax.experimental.pallas import tpu_sc as plsc

N = 10000
E = 160000
D = 256
H = 16
C = 64

BLK = 1000
NACC = 10240
KB = 128
NT = 32
EPT = 5120
NCHUNK = EPT // KB
RPT = NACC // 16
EPAD = NT * EPT
NBUF = 4


def _sc_mesh():
    return plsc.VectorSubcoreMesh(core_axis_name="c", subcore_axis_name="s")


def _stage_edges(wid, srcb, dstb, ewb, src_v, dst_v, ew_v):
    pltpu.sync_copy(srcb.at[wid], src_v)
    pltpu.sync_copy(dstb.at[wid], dst_v)
    pltpu.sync_copy(ewb.at[wid], ew_v)


def _agg_pipeline(F, c, src_v, dst_idx, ew_j, rows_v, acc, table_s, gsems,
                  ssems):
    def issue_gather(j, b):
        pltpu.async_copy(table_s.at[src_v.at[j]], rows_v.at[b], gsems[b])

    def wait_gather(j, b):
        pltpu.make_async_copy(table_s.at[src_v.at[j]], rows_v.at[b],
                              gsems[b]).wait()

    def issue_scatter(j, b):
        pltpu.async_copy(rows_v.at[b], acc.at[dst_idx(j)], ssems[b], add=True)

    def wait_scatter(j, b):
        pltpu.make_async_copy(rows_v.at[b], acc.at[dst_idx(j)],
                              ssems[b]).wait()

    def scale(j, b):
        def sbody(g, carry):
            wv = ew_j(j, g)
            for k in range(16):
                w = wv[k]
                i = g * 16 + k
                for f in range(F // 16):
                    sl = pl.ds(f * 16, 16)
                    rows_v[b, i, sl] = rows_v[b, i, sl] * w
            return carry

        lax.fori_loop(0, KB // 16, sbody, 0)

    for b in range(NBUF - 1):
        issue_gather(b, b)

    def step(j, b, bnext):
        wait_gather(j, b)
        scale(j, b)

        @pl.when(j > 0)
        def _():
            wait_scatter(j - 1, bnext)

        issue_scatter(j, b)

        @pl.when(j + NBUF - 1 < NCHUNK)
        def _():
            issue_gather(j + NBUF - 1, bnext)

    def body2(t, carry):
        for u in range(NBUF):
            step(NBUF * t + u, u, (u + NBUF - 1) % NBUF)
        return carry

    lax.fori_loop(0, NCHUNK // NBUF, body2, 0)
    wait_scatter(NCHUNK - 1, (NCHUNK - 1) % NBUF)


def _degagg16_body(xw1p, srcb, dstb, ewb, p_out, dinv_out, src_v, dst_v2,
                   ew_v2, rows_v, tloc, dloc, acc, table_s, deg_s, *sems):
    gsems = sems[:NBUF]
    ssems = sems[NBUF:2 * NBUF]
    dsem = sems[2 * NBUF]
    c = lax.axis_index("c")
    s = lax.axis_index("s")
    wid = s * 2 + c
    base = s * RPT

    pltpu.sync_copy(srcb.at[wid], src_v)
    pltpu.sync_copy(dstb.at[pl.ds(2 * s, 2)], dst_v2)
    pltpu.sync_copy(ewb.at[pl.ds(2 * s, 2)], ew_v2)
    pltpu.sync_copy(xw1p.at[pl.ds(base, RPT)], table_s.at[pl.ds(base, RPT)])

    def zbody(g, carry):
        dloc[pl.ds(g * 16, 16)] = jnp.zeros((16,), jnp.float32)
        return carry

    lax.fori_loop(0, RPT // 16, zbody, 0)
    pltpu.sync_copy(dloc, deg_s.at[pl.ds(base, RPT)])
    plsc.subcore_barrier()

    for q in range(2):
        def fire(j, carry):
            pltpu.async_copy(ew_v2.at[q, j], deg_s.at[dst_v2.at[q, j]], dsem,
                             add=True)
            return carry

        lax.fori_loop(0, NCHUNK, fire, 0)
    for q in range(2):
        def drain(j, carry):
            pltpu.make_async_copy(ew_v2.at[q, j], deg_s.at[dst_v2.at[q, j]],
                                  dsem).wait()
            return carry

        lax.fori_loop(0, NCHUNK, drain, 0)
    plsc.subcore_barrier()

    pltpu.sync_copy(deg_s.at[pl.ds(base, RPT)], dloc)
    pltpu.sync_copy(table_s.at[pl.ds(base, RPT)], tloc)

    def nbody(g, carry):
        sl = pl.ds(g * 16, 16)
        xdeg = dloc[sl] + 1.0
        bits = plsc.bitcast(xdeg, jnp.int32)
        y = plsc.bitcast(jnp.int32(0x5F3759DF) - (bits >> 1), jnp.float32)
        half = xdeg * 0.5
        y = y * (1.5 - half * y * y)
        y = y * (1.5 - half * y * y)
        y = y * (1.5 - half * y * y)
        dloc[sl] = y
        return carry

    lax.fori_loop(0, RPT // 16, nbody, 0)
    pltpu.sync_copy(dloc, dinv_out.at[c, pl.ds(base, RPT)])

    def tscale(g, carry):
        dv = dloc[pl.ds(g * 16, 16)]
        for k in range(16):
            i = g * 16 + k
            tloc[i, :] = tloc[i, :] * dv[k]
        return carry

    lax.fori_loop(0, RPT // 16, tscale, 0)
    pltpu.sync_copy(tloc, table_s.at[pl.ds(base, RPT)])
    pltpu.sync_copy(tloc, acc.at[pl.ds(base, RPT)])
    plsc.subcore_barrier()

    _agg_pipeline(H, c, src_v,
                  lambda j: dst_v2.at[c, j],
                  lambda j, g: ew_v2[c, j, pl.ds(g * 16, 16)],
                  rows_v, acc, table_s, gsems, ssems)
    plsc.subcore_barrier()
    pltpu.sync_copy(acc.at[pl.ds(base, RPT)], p_out.at[c, pl.ds(base, RPT)])


_degagg16 = functools.partial(
    pl.kernel,
    out_type=(
        jax.ShapeDtypeStruct((2, NACC, H), jnp.float32),
        jax.ShapeDtypeStruct((2, NACC), jnp.float32),
    ),
    mesh=_sc_mesh(),
    scratch_types=[
        pltpu.VMEM((NCHUNK, KB), jnp.int32),
        pltpu.VMEM((2, NCHUNK, KB), jnp.int32),
        pltpu.VMEM((2, NCHUNK, KB), jnp.float32),
        pltpu.VMEM((NBUF, KB, H), jnp.float32),
        pltpu.VMEM((RPT, H), jnp.float32),
        pltpu.VMEM((RPT,), jnp.float32),
        pltpu.VMEM_SHARED((NACC, H), jnp.float32),
        pltpu.VMEM_SHARED((NACC, H), jnp.float32),
        pltpu.VMEM_SHARED((NACC,), jnp.float32),
    ] + [pltpu.SemaphoreType.DMA] * (2 * NBUF + 1),
    compiler_params=pltpu.CompilerParams(use_tc_tiling_on_sc=False, needs_layout_passes=False),
)(_degagg16_body)


def _agg64_body(table_hbm, srcb, dstb, ewb, out_hbm, src_v, dst_v, ew_v,
                rows_v, acc, table_s, *sems):
    gsems = sems[:NBUF]
    ssems = sems[NBUF:]
    c = lax.axis_index("c")
    s = lax.axis_index("s")
    wid = s * 2 + c
    base = s * RPT
    _stage_edges(wid, srcb, dstb, ewb, src_v, dst_v, ew_v)
    pltpu.sync_copy(table_hbm.at[pl.ds(base, RPT)], acc.at[pl.ds(base, RPT)])
    pltpu.sync_copy(table_hbm.at[pl.ds(base, RPT)],
                    table_s.at[pl.ds(base, RPT)])
    plsc.subcore_barrier()
    _agg_pipeline(C, c, src_v,
                  lambda j: dst_v.at[j],
                  lambda j, g: ew_v[j, pl.ds(g * 16, 16)],
                  rows_v, acc, table_s, gsems, ssems)
    plsc.subcore_barrier()
    pltpu.sync_copy(acc.at[pl.ds(base, RPT)], out_hbm.at[c, pl.ds(base, RPT)])


_agg64 = functools.partial(
    pl.kernel,
    out_type=jax.ShapeDtypeStruct((2, NACC, C), jnp.float32),
    mesh=_sc_mesh(),
    scratch_types=[
        pltpu.VMEM((NCHUNK, KB), jnp.int32),
        pltpu.VMEM((NCHUNK, KB), jnp.int32),
        pltpu.VMEM((NCHUNK, KB), jnp.float32),
        pltpu.VMEM((NBUF, KB, C), jnp.float32),
        pltpu.VMEM_SHARED((NACC, C), jnp.float32),
        pltpu.VMEM_SHARED((NACC, C), jnp.float32),
    ] + [pltpu.SemaphoreType.DMA] * (2 * NBUF),
    compiler_params=pltpu.CompilerParams(use_tc_tiling_on_sc=False, needs_layout_passes=False),
)(_agg64_body)


def _mm1_body(x_ref, w_ref, xw_ref):
    xw_ref[...] = jnp.dot(x_ref[...], w_ref[...],
                          preferred_element_type=jnp.float32)


def _tc_mm1(x, W1):
    return pl.pallas_call(
        _mm1_body,
        grid=(N // BLK,),
        in_specs=[
            pl.BlockSpec((BLK, D), lambda i: (i, 0)),
            pl.BlockSpec((D, H), lambda i: (0, 0)),
        ],
        out_specs=pl.BlockSpec((BLK, H), lambda i: (i, 0)),
        out_shape=jax.ShapeDtypeStruct((N, H), jnp.float32),
    )(x, W1)


def _mm2_body(p0_ref, p1_ref, xw1_ref, dinv_ref, b1_ref, w2_ref, xs2_ref):
    dinv = dinv_ref[...]
    xs1 = xw1_ref[...] * dinv
    h = (p0_ref[0] + p1_ref[0] - xs1) * dinv + b1_ref[...]
    h = jnp.maximum(h, 0.0)
    xw2 = jnp.dot(h, w2_ref[...], preferred_element_type=jnp.float32)
    xs2_ref[...] = xw2 * dinv


def _tc_stage2(p, xw1, dinv, b1, W2):
    return pl.pallas_call(
        _mm2_body,
        grid=(N // BLK,),
        in_specs=[
            pl.BlockSpec((1, BLK, H), lambda i: (0, i, 0)),
            pl.BlockSpec((1, BLK, H), lambda i: (1, i, 0)),
            pl.BlockSpec((BLK, H), lambda i: (i, 0)),
            pl.BlockSpec((BLK, 1), lambda i: (i, 0)),
            pl.BlockSpec((1, H), lambda i: (0, 0)),
            pl.BlockSpec((H, C), lambda i: (0, 0)),
        ],
        out_specs=pl.BlockSpec((BLK, C), lambda i: (i, 0)),
        out_shape=jax.ShapeDtypeStruct((N, C), jnp.float32),
    )(p, p, xw1, dinv, b1, W2)


def _final_body(q0_ref, q1_ref, xs2_ref, dinv_ref, b2_ref, o_ref):
    o = (q0_ref[0] + q1_ref[0] - xs2_ref[...]) * dinv_ref[...] + b2_ref[...]
    m = jnp.max(o, axis=1, keepdims=True)
    ex = jnp.exp(o - m)
    sden = jnp.sum(ex, axis=1, keepdims=True)
    o_ref[...] = o - m - jnp.log(sden)


def _tc_final(q, xs2, dinv, b2):
    return pl.pallas_call(
        _final_body,
        grid=(N // BLK,),
        in_specs=[
            pl.BlockSpec((1, BLK, C), lambda i: (0, i, 0)),
            pl.BlockSpec((1, BLK, C), lambda i: (1, i, 0)),
            pl.BlockSpec((BLK, C), lambda i: (i, 0)),
            pl.BlockSpec((BLK, 1), lambda i: (i, 0)),
            pl.BlockSpec((1, C), lambda i: (0, 0)),
        ],
        out_specs=pl.BlockSpec((BLK, C), lambda i: (i, 0)),
        out_shape=jax.ShapeDtypeStruct((N, C), jnp.float32),
    )(q, q, xs2, dinv, b2)


def kernel(x, edge_index, edge_weight, W1, b1, W2, b2):
    src = edge_index[0]
    dst = edge_index[1]
    ew = edge_weight

    pad = EPAD - E
    srcb = jnp.concatenate([src, jnp.full((pad,), N, jnp.int32)]).reshape(NT, NCHUNK, KB)
    dstb = jnp.concatenate([dst, jnp.full((pad,), N, jnp.int32)]).reshape(NT, NCHUNK, KB)
    ewb = jnp.concatenate([ew, jnp.zeros((pad,), jnp.float32)]).reshape(NT, NCHUNK, KB)

    xw1 = _tc_mm1(x, W1)
    xw1p = jnp.zeros((NACC, H), jnp.float32).at[:N].set(xw1)
    p, dinvp = _degagg16(xw1p, srcb, dstb, ewb)

    dinv = dinvp[0, :N].reshape(N, 1)
    xs2 = _tc_stage2(p, xw1, dinv, b1.reshape(1, H), W2)

    xs2p = jnp.zeros((NACC, C), jnp.float32).at[:N].set(xs2)
    q = _agg64(xs2p, srcb, dstb, ewb)
    return _tc_final(q, xs2, dinv, b2.reshape(1, C))


# speedup vs baseline: 1.6113x; 1.0281x over previous
import functools

import jax
import jax.numpy as jnp
from jax import lax
from jax.experimental import pallas as pl
from jax.experimental.pallas import tpu as pltpu
from jax.experimental.pallas import tpu_sc as plsc

N = 10000
E = 160000
D = 256
H = 16
C = 64

BLK = 1000
NACC = 10240
KB = 128
NT = 32
EPT = 5120
NCHUNK = EPT // KB
RPT = NACC // 16
EPAD = NT * EPT
NBUF = 4


def _sc_mesh():
    return plsc.VectorSubcoreMesh(core_axis_name="c", subcore_axis_name="s")


def _stage_edges(wid, srcb, dstb, ewb, src_v, dst_v, ew_v):
    rows = pl.ds(wid * NCHUNK, NCHUNK)
    pltpu.sync_copy(srcb.at[rows], src_v)
    pltpu.sync_copy(dstb.at[rows], dst_v)
    pltpu.sync_copy(ewb.at[rows], ew_v)


def _agg_pipeline(F, c, src_v, dst_idx, ew_j, rows_v, acc, table_s, gsems,
                  ssems):
    def issue_gather(j, b):
        pltpu.async_copy(table_s.at[src_v.at[j]], rows_v.at[b], gsems[b])

    def wait_gather(j, b):
        pltpu.make_async_copy(table_s.at[src_v.at[j]], rows_v.at[b],
                              gsems[b]).wait()

    def issue_scatter(j, b):
        pltpu.async_copy(rows_v.at[b], acc.at[dst_idx(j)], ssems[b], add=True)

    def wait_scatter(j, b):
        pltpu.make_async_copy(rows_v.at[b], acc.at[dst_idx(j)],
                              ssems[b]).wait()

    def scale(j, b):
        def sbody(g, carry):
            wv = ew_j(j, g)
            for k in range(16):
                w = wv[k]
                i = g * 16 + k
                for f in range(F // 16):
                    sl = pl.ds(f * 16, 16)
                    rows_v[b, i, sl] = rows_v[b, i, sl] * w
            return carry

        lax.fori_loop(0, KB // 16, sbody, 0)

    for b in range(NBUF - 1):
        issue_gather(b, b)

    def step(j, b, bnext):
        wait_gather(j, b)
        scale(j, b)

        @pl.when(j > 0)
        def _():
            wait_scatter(j - 1, bnext)

        issue_scatter(j, b)

        @pl.when(j + NBUF - 1 < NCHUNK)
        def _():
            issue_gather(j + NBUF - 1, bnext)

    def body2(t, carry):
        for u in range(NBUF):
            step(NBUF * t + u, u, (u + NBUF - 1) % NBUF)
        return carry

    lax.fori_loop(0, NCHUNK // NBUF, body2, 0)
    wait_scatter(NCHUNK - 1, (NCHUNK - 1) % NBUF)


def _degagg16_body(xw1p, srcb, dstb, ewb, p_out, dinv_out, src_v, dst_v2,
                   ew_v2, rows_v, tloc, dloc, acc, table_s, deg_s, *sems):
    gsems = sems[:NBUF]
    ssems = sems[NBUF:2 * NBUF]
    dsem = sems[2 * NBUF]
    c = lax.axis_index("c")
    s = lax.axis_index("s")
    wid = s * 2 + c
    base = s * RPT

    pltpu.sync_copy(srcb.at[pl.ds(wid * NCHUNK, NCHUNK)], src_v)
    pltpu.sync_copy(dstb.at[pl.ds(2 * s * NCHUNK, 2 * NCHUNK)], dst_v2)
    pltpu.sync_copy(ewb.at[pl.ds(2 * s * NCHUNK, 2 * NCHUNK)], ew_v2)
    pltpu.sync_copy(xw1p.at[pl.ds(base, RPT)], table_s.at[pl.ds(base, RPT)])

    def zbody(g, carry):
        dloc[pl.ds(g * 16, 16)] = jnp.zeros((16,), jnp.float32)
        return carry

    lax.fori_loop(0, RPT // 16, zbody, 0)
    pltpu.sync_copy(dloc, deg_s.at[pl.ds(base, RPT)])
    plsc.subcore_barrier()

    for q in range(2):
        def fire(j, carry):
            pltpu.async_copy(ew_v2.at[q * NCHUNK + j],
                             deg_s.at[dst_v2.at[q * NCHUNK + j]], dsem,
                             add=True)
            return carry

        lax.fori_loop(0, NCHUNK, fire, 0)
    for q in range(2):
        def drain(j, carry):
            pltpu.make_async_copy(ew_v2.at[q * NCHUNK + j],
                                  deg_s.at[dst_v2.at[q * NCHUNK + j]],
                                  dsem).wait()
            return carry

        lax.fori_loop(0, NCHUNK, drain, 0)
    plsc.subcore_barrier()

    pltpu.sync_copy(deg_s.at[pl.ds(base, RPT)], dloc)
    pltpu.sync_copy(table_s.at[pl.ds(base, RPT)], tloc)

    def nbody(g, carry):
        sl = pl.ds(g * 16, 16)
        xdeg = dloc[sl] + 1.0
        bits = plsc.bitcast(xdeg, jnp.int32)
        y = plsc.bitcast(jnp.int32(0x5F3759DF) - (bits >> 1), jnp.float32)
        half = xdeg * 0.5
        y = y * (1.5 - half * y * y)
        y = y * (1.5 - half * y * y)
        y = y * (1.5 - half * y * y)
        dloc[sl] = y
        return carry

    lax.fori_loop(0, RPT // 16, nbody, 0)
    pltpu.sync_copy(dloc, dinv_out.at[c, pl.ds(base, RPT)])

    def tscale(g, carry):
        dv = dloc[pl.ds(g * 16, 16)]
        for k in range(16):
            i = g * 16 + k
            tloc[i, :] = tloc[i, :] * dv[k]
        return carry

    lax.fori_loop(0, RPT // 16, tscale, 0)
    pltpu.sync_copy(tloc, table_s.at[pl.ds(base, RPT)])
    pltpu.sync_copy(tloc, acc.at[pl.ds(base, RPT)])
    plsc.subcore_barrier()

    _agg_pipeline(H, c, src_v,
                  lambda j: dst_v2.at[c * NCHUNK + j],
                  lambda j, g: ew_v2[c * NCHUNK + j, pl.ds(g * 16, 16)],
                  rows_v, acc, table_s, gsems, ssems)
    plsc.subcore_barrier()
    pltpu.sync_copy(acc.at[pl.ds(base, RPT)], p_out.at[c, pl.ds(base, RPT)])


_degagg16 = functools.partial(
    pl.kernel,
    out_type=(
        jax.ShapeDtypeStruct((2, NACC, H), jnp.float32),
        jax.ShapeDtypeStruct((2, NACC), jnp.float32),
    ),
    mesh=_sc_mesh(),
    scratch_types=[
        pltpu.VMEM((NCHUNK, KB), jnp.int32),
        pltpu.VMEM((2 * NCHUNK, KB), jnp.int32),
        pltpu.VMEM((2 * NCHUNK, KB), jnp.float32),
        pltpu.VMEM((NBUF, KB, H), jnp.float32),
        pltpu.VMEM((RPT, H), jnp.float32),
        pltpu.VMEM((RPT,), jnp.float32),
        pltpu.VMEM_SHARED((NACC, H), jnp.float32),
        pltpu.VMEM_SHARED((NACC, H), jnp.float32),
        pltpu.VMEM_SHARED((NACC,), jnp.float32),
    ] + [pltpu.SemaphoreType.DMA] * (2 * NBUF + 1),
    compiler_params=pltpu.CompilerParams(use_tc_tiling_on_sc=False, needs_layout_passes=False),
)(_degagg16_body)


def _agg64_body(table_hbm, srcb, dstb, ewb, out_hbm, src_v, dst_v, ew_v,
                rows_v, acc, table_s, *sems):
    gsems = sems[:NBUF]
    ssems = sems[NBUF:]
    c = lax.axis_index("c")
    s = lax.axis_index("s")
    wid = s * 2 + c
    base = s * RPT
    _stage_edges(wid, srcb, dstb, ewb, src_v, dst_v, ew_v)
    pltpu.sync_copy(table_hbm.at[pl.ds(base, RPT)], acc.at[pl.ds(base, RPT)])
    pltpu.sync_copy(table_hbm.at[pl.ds(base, RPT)],
                    table_s.at[pl.ds(base, RPT)])
    plsc.subcore_barrier()
    _agg_pipeline(C, c, src_v,
                  lambda j: dst_v.at[j],
                  lambda j, g: ew_v[j, pl.ds(g * 16, 16)],
                  rows_v, acc, table_s, gsems, ssems)
    plsc.subcore_barrier()
    pltpu.sync_copy(acc.at[pl.ds(base, RPT)], out_hbm.at[c, pl.ds(base, RPT)])


_agg64 = functools.partial(
    pl.kernel,
    out_type=jax.ShapeDtypeStruct((2, NACC, C), jnp.float32),
    mesh=_sc_mesh(),
    scratch_types=[
        pltpu.VMEM((NCHUNK, KB), jnp.int32),
        pltpu.VMEM((NCHUNK, KB), jnp.int32),
        pltpu.VMEM((NCHUNK, KB), jnp.float32),
        pltpu.VMEM((NBUF, KB, C), jnp.float32),
        pltpu.VMEM_SHARED((NACC, C), jnp.float32),
        pltpu.VMEM_SHARED((NACC, C), jnp.float32),
    ] + [pltpu.SemaphoreType.DMA] * (2 * NBUF),
    compiler_params=pltpu.CompilerParams(use_tc_tiling_on_sc=False, needs_layout_passes=False),
)(_agg64_body)


def _mm1_body(x_ref, w_ref, xw_ref):
    xw_ref[:N] = jnp.dot(x_ref[...], w_ref[...],
                         preferred_element_type=jnp.float32)
    xw_ref[N:] = jnp.zeros((NACC - N, H), jnp.float32)


def _tc_mm1(x, W1):
    return pl.pallas_call(
        _mm1_body,
        out_shape=jax.ShapeDtypeStruct((NACC, H), jnp.float32),
    )(x, W1)


def _mm2_body(p_ref, xw1_ref, dinv_ref, b1_ref, w2_ref, xs2_ref):
    dinv = dinv_ref[:N]
    xs1 = xw1_ref[:N] * dinv
    h = (p_ref[0, :N] + p_ref[1, :N] - xs1) * dinv + b1_ref[...]
    h = jnp.maximum(h, 0.0)
    xw2 = jnp.dot(h, w2_ref[...], preferred_element_type=jnp.float32)
    xs2_ref[:N] = xw2 * dinv
    xs2_ref[N:] = jnp.zeros((NACC - N, C), jnp.float32)


def _tc_stage2(p, xw1, dinv, b1, W2):
    return pl.pallas_call(
        _mm2_body,
        out_shape=jax.ShapeDtypeStruct((NACC, C), jnp.float32),
    )(p, xw1, dinv, b1, W2)


def _final_body(q_ref, xs2_ref, dinv_ref, b2_ref, o_ref):
    o = ((q_ref[0, :N] + q_ref[1, :N] - xs2_ref[:N]) * dinv_ref[:N]
         + b2_ref[...])
    m = jnp.max(o, axis=1, keepdims=True)
    ex = jnp.exp(o - m)
    sden = jnp.sum(ex, axis=1, keepdims=True)
    o_ref[...] = o - m - jnp.log(sden)


def _tc_final(q, xs2, dinv, b2):
    return pl.pallas_call(
        _final_body,
        out_shape=jax.ShapeDtypeStruct((N, C), jnp.float32),
    )(q, xs2, dinv, b2)


def kernel(x, edge_index, edge_weight, W1, b1, W2, b2):
    src = edge_index[0]
    dst = edge_index[1]
    ew = edge_weight

    pad = EPAD - E
    srcb = jnp.concatenate([src, jnp.full((pad,), N, jnp.int32)]).reshape(NT * NCHUNK, KB)
    dstb = jnp.concatenate([dst, jnp.full((pad,), N, jnp.int32)]).reshape(NT * NCHUNK, KB)
    ewb = jnp.concatenate([ew, jnp.zeros((pad,), jnp.float32)]).reshape(NT * NCHUNK, KB)

    xw1p = _tc_mm1(x, W1)
    p, dinvp = _degagg16(xw1p, srcb, dstb, ewb)

    dinv = dinvp[0, :N].reshape(N, 1)
    xs2p = _tc_stage2(p, xw1p, dinv, b1.reshape(1, H), W2)

    q = _agg64(xs2p, srcb, dstb, ewb)
    return _tc_final(q, xs2p, dinv, b2.reshape(1, C))


# speedup vs baseline: 1.7781x; 1.1035x over previous
import functools

import jax
import jax.numpy as jnp
from jax import lax
from jax.experimental import pallas as pl
from jax.experimental.pallas import tpu as pltpu
from jax.experimental.pallas import tpu_sc as plsc

N = 10000
E = 160000
D = 256
H = 16
C = 64

BLK = 1000
NACC = 10240
KB = 128
NT = 32
EPT = 5120
NCHUNK = EPT // KB
RPT = NACC // 16
EPAD = NT * EPT
NBUF = 4


def _sc_mesh():
    return plsc.VectorSubcoreMesh(core_axis_name="c", subcore_axis_name="s")


def _stage_edges(wid, srcb, dstb, ewb, src_v, dst_v, ew_v):
    rows = pl.ds(wid * NCHUNK, NCHUNK)
    pltpu.sync_copy(srcb.at[rows], src_v)
    pltpu.sync_copy(dstb.at[rows], dst_v)
    pltpu.sync_copy(ewb.at[rows], ew_v)


def _agg_pipeline(F, c, src_v, dst_idx, ew_j, rows_v, acc, table_s, gsems,
                  ssems):
    def issue_gather(j, b):
        pltpu.async_copy(table_s.at[src_v.at[j]], rows_v.at[b], gsems[b])

    def wait_gather(j, b):
        pltpu.make_async_copy(table_s.at[src_v.at[j]], rows_v.at[b],
                              gsems[b]).wait()

    def issue_scatter(j, b):
        pltpu.async_copy(rows_v.at[b], acc.at[dst_idx(j)], ssems[b], add=True)

    def wait_scatter(j, b):
        pltpu.make_async_copy(rows_v.at[b], acc.at[dst_idx(j)],
                              ssems[b]).wait()

    def scale(j, b):
        def sbody(g, carry):
            wv = ew_j(j, g)
            for k in range(16):
                w = wv[k]
                i = g * 16 + k
                for f in range(F // 16):
                    sl = pl.ds(f * 16, 16)
                    rows_v[b, i, sl] = rows_v[b, i, sl] * w
            return carry

        lax.fori_loop(0, KB // 16, sbody, 0, unroll=KB // 16)

    for b in range(NBUF - 1):
        issue_gather(b, b)

    def step(j, b, bnext):
        wait_gather(j, b)
        scale(j, b)

        @pl.when(j > 0)
        def _():
            wait_scatter(j - 1, bnext)

        issue_scatter(j, b)

        @pl.when(j + NBUF - 1 < NCHUNK)
        def _():
            issue_gather(j + NBUF - 1, bnext)

    def body2(t, carry):
        for u in range(NBUF):
            step(NBUF * t + u, u, (u + NBUF - 1) % NBUF)
        return carry

    lax.fori_loop(0, NCHUNK // NBUF, body2, 0)
    wait_scatter(NCHUNK - 1, (NCHUNK - 1) % NBUF)


def _degagg16_body(xw1p, srcb, dstb, ewb, p_out, dinv_out, src_v, dst_v2,
                   ew_v2, rows_v, tloc, dloc, acc, table_s, deg_s, *sems):
    gsems = sems[:NBUF]
    ssems = sems[NBUF:2 * NBUF]
    dsem = sems[2 * NBUF]
    c = lax.axis_index("c")
    s = lax.axis_index("s")
    wid = s * 2 + c
    base = s * RPT

    pltpu.sync_copy(srcb.at[pl.ds(wid * NCHUNK, NCHUNK)], src_v)
    pltpu.sync_copy(dstb.at[pl.ds(2 * s * NCHUNK, 2 * NCHUNK)], dst_v2)
    pltpu.sync_copy(ewb.at[pl.ds(2 * s * NCHUNK, 2 * NCHUNK)], ew_v2)
    pltpu.sync_copy(xw1p.at[pl.ds(base, RPT)], table_s.at[pl.ds(base, RPT)])

    def zbody(g, carry):
        dloc[pl.ds(g * 16, 16)] = jnp.zeros((16,), jnp.float32)
        return carry

    lax.fori_loop(0, RPT // 16, zbody, 0)
    pltpu.sync_copy(dloc, deg_s.at[pl.ds(base, RPT)])
    plsc.subcore_barrier()

    for q in range(2):
        def fire(j, carry):
            pltpu.async_copy(ew_v2.at[q * NCHUNK + j],
                             deg_s.at[dst_v2.at[q * NCHUNK + j]], dsem,
                             add=True)
            return carry

        lax.fori_loop(0, NCHUNK, fire, 0)
    for q in range(2):
        def drain(j, carry):
            pltpu.make_async_copy(ew_v2.at[q * NCHUNK + j],
                                  deg_s.at[dst_v2.at[q * NCHUNK + j]],
                                  dsem).wait()
            return carry

        lax.fori_loop(0, NCHUNK, drain, 0)
    plsc.subcore_barrier()

    pltpu.sync_copy(deg_s.at[pl.ds(base, RPT)], dloc)
    pltpu.sync_copy(table_s.at[pl.ds(base, RPT)], tloc)

    def nbody(g, carry):
        sl = pl.ds(g * 16, 16)
        xdeg = dloc[sl] + 1.0
        bits = plsc.bitcast(xdeg, jnp.int32)
        y = plsc.bitcast(jnp.int32(0x5F3759DF) - (bits >> 1), jnp.float32)
        half = xdeg * 0.5
        y = y * (1.5 - half * y * y)
        y = y * (1.5 - half * y * y)
        y = y * (1.5 - half * y * y)
        dloc[sl] = y
        return carry

    lax.fori_loop(0, RPT // 16, nbody, 0)
    pltpu.sync_copy(dloc, dinv_out.at[c, pl.ds(base, RPT)])

    def tscale(g, carry):
        dv = dloc[pl.ds(g * 16, 16)]
        for k in range(16):
            i = g * 16 + k
            tloc[i, :] = tloc[i, :] * dv[k]
        return carry

    lax.fori_loop(0, RPT // 16, tscale, 0)
    pltpu.sync_copy(tloc, table_s.at[pl.ds(base, RPT)])
    pltpu.sync_copy(tloc, acc.at[pl.ds(base, RPT)])
    plsc.subcore_barrier()

    _agg_pipeline(H, c, src_v,
                  lambda j: dst_v2.at[c * NCHUNK + j],
                  lambda j, g: ew_v2[c * NCHUNK + j, pl.ds(g * 16, 16)],
                  rows_v, acc, table_s, gsems, ssems)
    plsc.subcore_barrier()
    pltpu.sync_copy(acc.at[pl.ds(base, RPT)], p_out.at[c, pl.ds(base, RPT)])


_degagg16 = functools.partial(
    pl.kernel,
    out_type=(
        jax.ShapeDtypeStruct((2, NACC, H), jnp.float32),
        jax.ShapeDtypeStruct((2, NACC), jnp.float32),
    ),
    mesh=_sc_mesh(),
    scratch_types=[
        pltpu.VMEM((NCHUNK, KB), jnp.int32),
        pltpu.VMEM((2 * NCHUNK, KB), jnp.int32),
        pltpu.VMEM((2 * NCHUNK, KB), jnp.float32),
        pltpu.VMEM((NBUF, KB, H), jnp.float32),
        pltpu.VMEM((RPT, H), jnp.float32),
        pltpu.VMEM((RPT,), jnp.float32),
        pltpu.VMEM_SHARED((NACC, H), jnp.float32),
        pltpu.VMEM_SHARED((NACC, H), jnp.float32),
        pltpu.VMEM_SHARED((NACC,), jnp.float32),
    ] + [pltpu.SemaphoreType.DMA] * (2 * NBUF + 1),
    compiler_params=pltpu.CompilerParams(use_tc_tiling_on_sc=False, needs_layout_passes=False),
)(_degagg16_body)


def _agg64_body(table_hbm, srcb, dstb, ewb, out_hbm, src_v, dst_v, ew_v,
                rows_v, acc, table_s, *sems):
    gsems = sems[:NBUF]
    ssems = sems[NBUF:]
    c = lax.axis_index("c")
    s = lax.axis_index("s")
    wid = s * 2 + c
    base = s * RPT
    _stage_edges(wid, srcb, dstb, ewb, src_v, dst_v, ew_v)
    pltpu.sync_copy(table_hbm.at[pl.ds(base, RPT)], acc.at[pl.ds(base, RPT)])
    pltpu.sync_copy(table_hbm.at[pl.ds(base, RPT)],
                    table_s.at[pl.ds(base, RPT)])
    plsc.subcore_barrier()
    _agg_pipeline(C, c, src_v,
                  lambda j: dst_v.at[j],
                  lambda j, g: ew_v[j, pl.ds(g * 16, 16)],
                  rows_v, acc, table_s, gsems, ssems)
    plsc.subcore_barrier()
    pltpu.sync_copy(acc.at[pl.ds(base, RPT)], out_hbm.at[c, pl.ds(base, RPT)])


_agg64 = functools.partial(
    pl.kernel,
    out_type=jax.ShapeDtypeStruct((2, NACC, C), jnp.float32),
    mesh=_sc_mesh(),
    scratch_types=[
        pltpu.VMEM((NCHUNK, KB), jnp.int32),
        pltpu.VMEM((NCHUNK, KB), jnp.int32),
        pltpu.VMEM((NCHUNK, KB), jnp.float32),
        pltpu.VMEM((NBUF, KB, C), jnp.float32),
        pltpu.VMEM_SHARED((NACC, C), jnp.float32),
        pltpu.VMEM_SHARED((NACC, C), jnp.float32),
    ] + [pltpu.SemaphoreType.DMA] * (2 * NBUF),
    compiler_params=pltpu.CompilerParams(use_tc_tiling_on_sc=False, needs_layout_passes=False),
)(_agg64_body)


def _mm1_body(x_ref, w_ref, xw_ref):
    xw_ref[:N] = jnp.dot(x_ref[...], w_ref[...],
                         preferred_element_type=jnp.float32)
    xw_ref[N:] = jnp.zeros((NACC - N, H), jnp.float32)


def _tc_mm1(x, W1):
    return pl.pallas_call(
        _mm1_body,
        out_shape=jax.ShapeDtypeStruct((NACC, H), jnp.float32),
    )(x, W1)


def _mm2_body(p_ref, xw1_ref, dinv_ref, b1_ref, w2_ref, xs2_ref):
    dinv = dinv_ref[:N]
    xs1 = xw1_ref[:N] * dinv
    h = (p_ref[0, :N] + p_ref[1, :N] - xs1) * dinv + b1_ref[...]
    h = jnp.maximum(h, 0.0)
    xw2 = jnp.dot(h, w2_ref[...], preferred_element_type=jnp.float32)
    xs2_ref[:N] = xw2 * dinv
    xs2_ref[N:] = jnp.zeros((NACC - N, C), jnp.float32)


def _tc_stage2(p, xw1, dinv, b1, W2):
    return pl.pallas_call(
        _mm2_body,
        out_shape=jax.ShapeDtypeStruct((NACC, C), jnp.float32),
    )(p, xw1, dinv, b1, W2)


def _final_body(q_ref, xs2_ref, dinv_ref, b2_ref, o_ref):
    o = ((q_ref[0, :N] + q_ref[1, :N] - xs2_ref[:N]) * dinv_ref[:N]
         + b2_ref[...])
    m = jnp.max(o, axis=1, keepdims=True)
    ex = jnp.exp(o - m)
    sden = jnp.sum(ex, axis=1, keepdims=True)
    o_ref[...] = o - m - jnp.log(sden)


def _tc_final(q, xs2, dinv, b2):
    return pl.pallas_call(
        _final_body,
        out_shape=jax.ShapeDtypeStruct((N, C), jnp.float32),
    )(q, xs2, dinv, b2)


def kernel(x, edge_index, edge_weight, W1, b1, W2, b2):
    src = edge_index[0]
    dst = edge_index[1]
    ew = edge_weight

    pad = EPAD - E
    srcb = jnp.concatenate([src, jnp.full((pad,), N, jnp.int32)]).reshape(NT * NCHUNK, KB)
    dstb = jnp.concatenate([dst, jnp.full((pad,), N, jnp.int32)]).reshape(NT * NCHUNK, KB)
    ewb = jnp.concatenate([ew, jnp.zeros((pad,), jnp.float32)]).reshape(NT * NCHUNK, KB)

    xw1p = _tc_mm1(x, W1)
    p, dinvp = _degagg16(xw1p, srcb, dstb, ewb)

    dinv = dinvp[0, :N].reshape(N, 1)
    xs2p = _tc_stage2(p, xw1p, dinv, b1.reshape(1, H), W2)

    q = _agg64(xs2p, srcb, dstb, ewb)
    return _tc_final(q, xs2p, dinv, b2.reshape(1, C))


# speedup vs baseline: 1.9247x; 1.0825x over previous
import functools

import jax
import jax.numpy as jnp
from jax import lax
from jax.experimental import pallas as pl
from jax.experimental.pallas import tpu as pltpu
from jax.experimental.pallas import tpu_sc as plsc

N = 10000
E = 160000
D = 256
H = 16
C = 64

BLK = 1000
NACC = 10240
KB = 128
NT = 32
NCH_ALL = E // KB
NCHA = 40
DCH = 80
RPT = NACC // 16
NBUF = 4


def _sc_mesh():
    return plsc.VectorSubcoreMesh(core_axis_name="c", subcore_axis_name="s")


def _chunk_range(wid):
    start = 39 * wid + jnp.minimum(wid, 2)
    cnt = jnp.where(wid < 2, NCHA, NCHA - 1)
    return start, cnt


def _agg_pipeline(F, cnt, src_idx, dst_idx, ew_j, rows_v, acc, table_s,
                  gsems, ssems):
    def issue_gather(j, b):
        pltpu.async_copy(table_s.at[src_idx(j)], rows_v.at[b], gsems[b])

    def wait_gather(j, b):
        pltpu.make_async_copy(table_s.at[src_idx(j)], rows_v.at[b],
                              gsems[b]).wait()

    def issue_scatter(j, b):
        pltpu.async_copy(rows_v.at[b], acc.at[dst_idx(j)], ssems[b], add=True)

    def wait_scatter(j, b):
        pltpu.make_async_copy(rows_v.at[b], acc.at[dst_idx(j)],
                              ssems[b]).wait()

    def scale(j, b):
        def sbody(g, carry):
            wv = ew_j(j, g)
            for k in range(16):
                w = wv[k]
                i = g * 16 + k
                for f in range(F // 16):
                    sl = pl.ds(f * 16, 16)
                    rows_v[b, i, sl] = rows_v[b, i, sl] * w
            return carry

        lax.fori_loop(0, KB // 16, sbody, 0, unroll=KB // 16)

    for b in range(NBUF - 1):
        issue_gather(b, b)

    def step(j, b, bnext):
        @pl.when(j < cnt)
        def _():
            wait_gather(j, b)
            scale(j, b)

            @pl.when(j > 0)
            def _():
                wait_scatter(j - 1, bnext)

            issue_scatter(j, b)

            @pl.when(j + NBUF - 1 < cnt)
            def _():
                issue_gather(j + NBUF - 1, bnext)

    def body2(t, carry):
        for u in range(NBUF):
            step(NBUF * t + u, u, (u + NBUF - 1) % NBUF)
        return carry

    lax.fori_loop(0, NCHA // NBUF, body2, 0)

    @pl.when(cnt == NCHA)
    def _():
        wait_scatter(NCHA - 1, (NCHA - 1) % NBUF)

    @pl.when(cnt == NCHA - 1)
    def _():
        wait_scatter(NCHA - 2, (NCHA - 2) % NBUF)


def _degagg16_body(xw1p, eib, ewb, p_out, dinv_out, src_v, dst_v2,
                   ew_v2, rows_v, tloc, dloc, acc, table_s, deg_s, *sems):
    gsems = sems[:NBUF]
    ssems = sems[NBUF:2 * NBUF]
    dsem = sems[2 * NBUF]
    c = lax.axis_index("c")
    s = lax.axis_index("s")
    wid = s * 2 + c
    base = s * RPT

    start_a, cnt_a = _chunk_range(wid)
    start_d, _ = _chunk_range(2 * s)
    cnt_d = jnp.where(s < 1, DCH, DCH - 2)
    stage_d = jnp.minimum(start_d, NCH_ALL - DCH)
    loff_d = start_d - stage_d
    stage_s = jnp.minimum(start_a, NCH_ALL - NCHA)
    loff_s = start_a - stage_s
    loff_a = start_a - stage_d
    pltpu.sync_copy(eib.at[0, pl.ds(stage_s, NCHA)], src_v)
    pltpu.sync_copy(eib.at[1, pl.ds(stage_d, DCH)], dst_v2)
    pltpu.sync_copy(ewb.at[pl.ds(stage_d, DCH)], ew_v2)
    pltpu.sync_copy(xw1p.at[pl.ds(base, RPT)], table_s.at[pl.ds(base, RPT)])

    def zbody(g, carry):
        dloc[pl.ds(g * 16, 16)] = jnp.zeros((16,), jnp.float32)
        return carry

    lax.fori_loop(0, RPT // 16, zbody, 0)
    pltpu.sync_copy(dloc, deg_s.at[pl.ds(base, RPT)])
    plsc.subcore_barrier()

    def fire(j, carry):
        pltpu.async_copy(ew_v2.at[loff_d + j], deg_s.at[dst_v2.at[loff_d + j]],
                         dsem, add=True)
        return carry

    lax.fori_loop(0, cnt_d, fire, 0)

    def drain(j, carry):
        pltpu.make_async_copy(ew_v2.at[loff_d + j],
                              deg_s.at[dst_v2.at[loff_d + j]], dsem).wait()
        return carry

    lax.fori_loop(0, cnt_d, drain, 0)
    plsc.subcore_barrier()

    pltpu.sync_copy(deg_s.at[pl.ds(base, RPT)], dloc)
    pltpu.sync_copy(table_s.at[pl.ds(base, RPT)], tloc)

    def nbody(g, carry):
        sl = pl.ds(g * 16, 16)
        xdeg = dloc[sl] + 1.0
        bits = plsc.bitcast(xdeg, jnp.int32)
        y = plsc.bitcast(jnp.int32(0x5F3759DF) - (bits >> 1), jnp.float32)
        half = xdeg * 0.5
        y = y * (1.5 - half * y * y)
        y = y * (1.5 - half * y * y)
        y = y * (1.5 - half * y * y)
        dloc[sl] = y
        return carry

    lax.fori_loop(0, RPT // 16, nbody, 0)
    pltpu.sync_copy(dloc, dinv_out.at[c, pl.ds(base, RPT)])

    def tscale(g, carry):
        dv = dloc[pl.ds(g * 16, 16)]
        for k in range(16):
            i = g * 16 + k
            tloc[i, :] = tloc[i, :] * dv[k]
        return carry

    lax.fori_loop(0, RPT // 16, tscale, 0)
    pltpu.sync_copy(tloc, table_s.at[pl.ds(base, RPT)])
    pltpu.sync_copy(tloc, acc.at[pl.ds(base, RPT)])
    plsc.subcore_barrier()

    _agg_pipeline(H, cnt_a,
                  lambda j: src_v.at[loff_s + j],
                  lambda j: dst_v2.at[loff_a + j],
                  lambda j, g: ew_v2[loff_a + j, pl.ds(g * 16, 16)],
                  rows_v, acc, table_s, gsems, ssems)
    plsc.subcore_barrier()
    pltpu.sync_copy(acc.at[pl.ds(base, RPT)], p_out.at[c, pl.ds(base, RPT)])


_degagg16 = functools.partial(
    pl.kernel,
    out_type=(
        jax.ShapeDtypeStruct((2, NACC, H), jnp.float32),
        jax.ShapeDtypeStruct((2, NACC), jnp.float32),
    ),
    mesh=_sc_mesh(),
    scratch_types=[
        pltpu.VMEM((NCHA, KB), jnp.int32),
        pltpu.VMEM((DCH, KB), jnp.int32),
        pltpu.VMEM((DCH, KB), jnp.float32),
        pltpu.VMEM((NBUF, KB, H), jnp.float32),
        pltpu.VMEM((RPT, H), jnp.float32),
        pltpu.VMEM((RPT,), jnp.float32),
        pltpu.VMEM_SHARED((NACC, H), jnp.float32),
        pltpu.VMEM_SHARED((NACC, H), jnp.float32),
        pltpu.VMEM_SHARED((NACC,), jnp.float32),
    ] + [pltpu.SemaphoreType.DMA] * (2 * NBUF + 1),
    compiler_params=pltpu.CompilerParams(use_tc_tiling_on_sc=False, needs_layout_passes=False),
)(_degagg16_body)


def _agg64_body(table_hbm, eib, ewb, out_hbm, src_v, dst_v, ew_v,
                rows_v, acc, table_s, *sems):
    gsems = sems[:NBUF]
    ssems = sems[NBUF:]
    c = lax.axis_index("c")
    s = lax.axis_index("s")
    wid = s * 2 + c
    base = s * RPT
    start_a, cnt_a = _chunk_range(wid)
    stage_s = jnp.minimum(start_a, NCH_ALL - NCHA)
    loff = start_a - stage_s
    pltpu.sync_copy(eib.at[0, pl.ds(stage_s, NCHA)], src_v)
    pltpu.sync_copy(eib.at[1, pl.ds(stage_s, NCHA)], dst_v)
    pltpu.sync_copy(ewb.at[pl.ds(stage_s, NCHA)], ew_v)
    pltpu.sync_copy(table_hbm.at[pl.ds(base, RPT)], acc.at[pl.ds(base, RPT)])
    pltpu.sync_copy(table_hbm.at[pl.ds(base, RPT)],
                    table_s.at[pl.ds(base, RPT)])
    plsc.subcore_barrier()
    _agg_pipeline(C, cnt_a,
                  lambda j: src_v.at[loff + j],
                  lambda j: dst_v.at[loff + j],
                  lambda j, g: ew_v[loff + j, pl.ds(g * 16, 16)],
                  rows_v, acc, table_s, gsems, ssems)
    plsc.subcore_barrier()
    pltpu.sync_copy(acc.at[pl.ds(base, RPT)], out_hbm.at[c, pl.ds(base, RPT)])


_agg64 = functools.partial(
    pl.kernel,
    out_type=jax.ShapeDtypeStruct((2, NACC, C), jnp.float32),
    mesh=_sc_mesh(),
    scratch_types=[
        pltpu.VMEM((NCHA, KB), jnp.int32),
        pltpu.VMEM((NCHA, KB), jnp.int32),
        pltpu.VMEM((NCHA, KB), jnp.float32),
        pltpu.VMEM((NBUF, KB, C), jnp.float32),
        pltpu.VMEM_SHARED((NACC, C), jnp.float32),
        pltpu.VMEM_SHARED((NACC, C), jnp.float32),
    ] + [pltpu.SemaphoreType.DMA] * (2 * NBUF),
    compiler_params=pltpu.CompilerParams(use_tc_tiling_on_sc=False, needs_layout_passes=False),
)(_agg64_body)


def _mm1_body(x_ref, w_ref, xw_ref):
    xw_ref[:N] = jnp.dot(x_ref[...], w_ref[...],
                         preferred_element_type=jnp.float32)
    xw_ref[N:] = jnp.zeros((NACC - N, H), jnp.float32)


def _tc_mm1(x, W1):
    return pl.pallas_call(
        _mm1_body,
        out_shape=jax.ShapeDtypeStruct((NACC, H), jnp.float32),
    )(x, W1)


def _mm2_body(p_ref, xw1_ref, dinv_ref, b1_ref, w2_ref, xs2_ref):
    dinv = dinv_ref[:N]
    xs1 = xw1_ref[:N] * dinv
    h = (p_ref[0, :N] + p_ref[1, :N] - xs1) * dinv + b1_ref[...]
    h = jnp.maximum(h, 0.0)
    xw2 = jnp.dot(h, w2_ref[...], preferred_element_type=jnp.float32)
    xs2_ref[:N] = xw2 * dinv
    xs2_ref[N:] = jnp.zeros((NACC - N, C), jnp.float32)


def _tc_stage2(p, xw1, dinv, b1, W2):
    return pl.pallas_call(
        _mm2_body,
        out_shape=jax.ShapeDtypeStruct((NACC, C), jnp.float32),
    )(p, xw1, dinv, b1, W2)


def _final_body(q_ref, xs2_ref, dinv_ref, b2_ref, o_ref):
    o = ((q_ref[0, :N] + q_ref[1, :N] - xs2_ref[:N]) * dinv_ref[:N]
         + b2_ref[...])
    m = jnp.max(o, axis=1, keepdims=True)
    ex = jnp.exp(o - m)
    sden = jnp.sum(ex, axis=1, keepdims=True)
    o_ref[...] = o - m - jnp.log(sden)


def _tc_final(q, xs2, dinv, b2):
    return pl.pallas_call(
        _final_body,
        out_shape=jax.ShapeDtypeStruct((N, C), jnp.float32),
    )(q, xs2, dinv, b2)


def kernel(x, edge_index, edge_weight, W1, b1, W2, b2):
    eib = edge_index.reshape(2, NCH_ALL, KB)
    ewb = edge_weight.reshape(NCH_ALL, KB)

    xw1p = _tc_mm1(x, W1)
    p, dinvp = _degagg16(xw1p, eib, ewb)

    dinv = dinvp[0, :N].reshape(N, 1)
    xs2p = _tc_stage2(p, xw1p, dinv, b1.reshape(1, H), W2)

    q = _agg64(xs2p, eib, ewb)
    return _tc_final(q, xs2p, dinv, b2.reshape(1, C))
